# baseline (speedup 1.0000x reference)
import functools

import jax
import jax.numpy as jnp
from jax import lax
from jax.experimental import pallas as pl
from jax.experimental.pallas import tpu as pltpu
from jax.experimental.pallas import tpu_sc as plsc

_IN_DIM = 100000
_OUT_DIM = 64
_PAD_DIM = 128

_NC = 2
_NS = 16
_NW = _NC * _NS
_LANES = 16
_SUB = 8

_B = 4096 * 200
_BPW = _B // _NW
_CH = 128
_NCH = _BPW // _CH
_CHT = _CH // _SUB


def _sc_lookup(table_hbm, idx_hbm, out_hbm, idx_v, bufa0, bufa1,
               bufb0, bufb1, sem_i, sg0, sg1, ss0, ss1):
    wid = lax.axis_index("s") * _NC + lax.axis_index("c")
    base_t = wid * (_BPW // _SUB)

    pltpu.async_copy(idx_hbm.at[wid], idx_v, sem_i).wait()

    bufa = (bufa0, bufa1)
    bufb = (bufb0, bufb1)
    sgs = (sg0, sg1)
    sss = (ss0, ss1)

    def start_gather(g, p):
        for k in range(_CH // _LANES):
            v = idx_v[g, pl.ds(k * _LANES, _LANES)]
            v = jnp.minimum(jnp.maximum(v, -_IN_DIM), _IN_DIM) + _IN_DIM
            idx_v[g, pl.ds(k * _LANES, _LANES)] = v
        pltpu.async_copy(table_hbm.at[idx_v.at[g]], bufa[p], sgs[p])

    def wait_gather(p):
        pltpu.make_async_copy(
            table_hbm.at[pl.ds(0, _CH)], bufa[p], sgs[p]).wait()

    def compact(p):
        def rows(t, carry):
            for s in range(_SUB):
                for k in range(_OUT_DIM // _LANES):
                    bufb[p][t, s, pl.ds(k * _LANES, _LANES)] = (
                        bufa[p][t * _SUB + s, pl.ds(k * _LANES, _LANES)])
            return carry

        lax.fori_loop(0, _CHT, rows, 0)

    def start_scatter(g, p):
        pltpu.async_copy(
            bufb[p], out_hbm.at[pl.ds(base_t + g * _CHT, _CHT)], sss[p])

    def wait_scatter(p):
        pltpu.make_async_copy(
            bufb[p], out_hbm.at[pl.ds(0, _CHT)], sss[p]).wait()

    def step(g, p, first, last):
        wait_gather(p)
        if not first:
            wait_scatter(p)
        if not last:
            start_gather(g + 1, 1 - p)
        compact(p)
        start_scatter(g, p)

    start_gather(0, 0)

    def body(gg, carry):
        g0 = 2 * gg

        @pl.when(gg == 0)
        def _():
            step(g0, 0, first=True, last=False)
            step(g0 + 1, 1, first=True, last=False)

        @pl.when(jnp.logical_and(gg > 0, gg < _NCH // 2 - 1))
        def _():
            step(g0, 0, first=False, last=False)
            step(g0 + 1, 1, first=False, last=False)

        @pl.when(gg == _NCH // 2 - 1)
        def _():
            step(g0, 0, first=False, last=False)
            step(g0 + 1, 1, first=False, last=True)

        return carry

    lax.fori_loop(0, _NCH // 2, body, 0)
    wait_scatter(0)
    wait_scatter(1)


def kernel(inputs, embeddings):
    idx = inputs.astype(jnp.int32).reshape(_NW, _NCH, _CH)
    table = jnp.pad(embeddings, ((0, 0), (0, _PAD_DIM - _OUT_DIM)))
    mesh = plsc.VectorSubcoreMesh(core_axis_name="c", subcore_axis_name="s")
    call = functools.partial(
        pl.kernel,
        mesh=mesh,
        out_type=jax.ShapeDtypeStruct((_B // _SUB, _SUB, _OUT_DIM),
                                      jnp.float32),
        scratch_types=[
            pltpu.VMEM((_NCH, _CH), jnp.int32),
            pltpu.VMEM((_CH, _PAD_DIM), jnp.float32),
            pltpu.VMEM((_CH, _PAD_DIM), jnp.float32),
            pltpu.VMEM((_CHT, _SUB, _OUT_DIM), jnp.float32),
            pltpu.VMEM((_CHT, _SUB, _OUT_DIM), jnp.float32),
            pltpu.SemaphoreType.DMA,
            pltpu.SemaphoreType.DMA,
            pltpu.SemaphoreType.DMA,
            pltpu.SemaphoreType.DMA,
            pltpu.SemaphoreType.DMA,
        ],
        compiler_params=pltpu.CompilerParams(use_tc_tiling_on_sc=True),
    )(_sc_lookup)
    out = call(table, idx)
    return out.reshape(inputs.shape[0], inputs.shape[1], _OUT_DIM)
